# baseline (speedup 1.0000x reference)
import functools

import jax
import jax.numpy as jnp
from jax import lax
from jax.experimental import pallas as pl
from jax.experimental.pallas import tpu as pltpu
from jax.experimental.pallas import tpu_sc as plsc

N_NODES = 10000
N_EDGES = 320000
D = 128

NC = 2
NS = 16
NW = NC * NS
EPW = N_EDGES // NW
CH = 112
EPAD = 10080
NCHUNK = EPAD // CH
RING = 3
NGRP = -(-NCHUNK // RING)
RPT = 624
RTAIL = N_NODES - NS * RPT


def _spmm_body(sup_hbm, edata_hbm, zeros_hbm, part_hbm,
               acc, ebuf, rows_v, esem, gsem, ssem):
    c = lax.axis_index("c")
    s = lax.axis_index("s")
    wid = s * NC + c
    my_edata = edata_hbm.at[wid]

    pltpu.sync_copy(zeros_hbm.at[pl.ds(s * RPT, RPT)],
                    acc.at[pl.ds(s * RPT, RPT)])

    @pl.when(s == NS - 1)
    def _zero_tail():
        tl = pl.ds(NS * RPT, RTAIL)
        pltpu.sync_copy(zeros_hbm.at[tl], acc.at[tl])

    plsc.subcore_barrier()

    def e_issue(cj, k):
        pltpu.async_copy(my_edata.at[cj], ebuf.at[pl.ds(3 * k, 3)],
                         esem.at[k])

    def e_wait(k):
        pltpu.make_async_copy(my_edata.at[0], ebuf.at[pl.ds(3 * k, 3)],
                              esem.at[k]).wait()

    def rows_slot(k):
        return rows_v.at[pl.ds(k * CH, CH)]

    def g_issue(k):
        pltpu.async_copy(sup_hbm.at[ebuf.at[3 * k]], rows_slot(k),
                         gsem.at[k])

    def g_wait(k):
        pltpu.make_async_copy(sup_hbm.at[ebuf.at[3 * k]], rows_slot(k),
                              gsem.at[k]).wait()

    def s_issue(k):
        pltpu.async_copy(rows_slot(k), acc.at[ebuf.at[3 * k + 1]],
                         ssem.at[k], add=True)

    def s_wait(k):
        pltpu.make_async_copy(rows_slot(k), acc.at[ebuf.at[3 * k + 1]],
                              ssem.at[k]).wait()

    def scale_slot(k):
        @plsc.parallel_loop(0, CH, unroll=8)
        def edge_body(e):
            wbits = plsc.load_gather(
                ebuf, [jnp.full((16,), 3 * k + 2, dtype=jnp.int32),
                       jnp.full((16,), e, dtype=jnp.int32)])
            wvec = plsc.bitcast(wbits, jnp.float32)
            row = k * CH + e
            for j in range(D // 16):
                sl = pl.ds(j * 16, 16)
                rows_v[row, sl] = rows_v[row, sl] * wvec

    e_issue(0, 0)
    e_issue(1, 1)
    e_wait(0)
    g_issue(0)

    def group_body(g, _):
        c0 = g * RING
        for k in range(RING):
            cj = c0 + k
            nb1 = (k + 1) % RING
            nb2 = (k + 2) % RING

            @pl.when(cj + 2 < NCHUNK)
            def _prefetch_edata():
                @pl.when(cj + 2 >= RING)
                def _drain_scatter():
                    s_wait(nb2)
                e_issue(cj + 2, nb2)

            @pl.when(cj + 1 < NCHUNK)
            def _prefetch_gather():
                e_wait(nb1)
                g_issue(nb1)

            @pl.when(cj < NCHUNK)
            def _process():
                g_wait(k)
                scale_slot(k)
                s_issue(k)
        return 0

    lax.fori_loop(0, NGRP, group_body, 0, unroll=False)
    for k in range(RING):
        s_wait(k)
    plsc.subcore_barrier()

    sl = pl.ds(s * RPT, RPT)
    pltpu.sync_copy(acc.at[sl], part_hbm.at[c].at[sl])

    @pl.when(s == NS - 1)
    def _write_tail():
        tl = pl.ds(NS * RPT, RTAIL)
        pltpu.sync_copy(acc.at[tl], part_hbm.at[c].at[tl])


@functools.lru_cache(maxsize=None)
def _make_spmm():
    return pl.kernel(
        _spmm_body,
        out_type=jax.ShapeDtypeStruct((NC, N_NODES, D), jnp.float32),
        mesh=plsc.VectorSubcoreMesh(core_axis_name="c", subcore_axis_name="s"),
        compiler_params=pltpu.CompilerParams(needs_layout_passes=False),
        scratch_types=[
            pltpu.VMEM_SHARED((N_NODES, D), jnp.float32),
            pltpu.VMEM((RING * 3, CH), jnp.int32),
            pltpu.VMEM((RING * CH, D), jnp.float32),
            pltpu.SemaphoreType.DMA((RING,)),
            pltpu.SemaphoreType.DMA((RING,)),
            pltpu.SemaphoreType.DMA((RING,)),
        ],
    )


_BLK = 1000


def _mm_bias_kernel(p_ref, w_ref, b_ref, o_ref):
    o_ref[...] = jnp.dot(p_ref[0] + p_ref[1], w_ref[...],
                         preferred_element_type=jnp.float32,
                         precision=lax.Precision.HIGHEST) + b_ref[...]


def _mm_bias(parts, w, b2d):
    return pl.pallas_call(
        _mm_bias_kernel,
        grid=(N_NODES // _BLK,),
        in_specs=[
            pl.BlockSpec((NC, _BLK, D), lambda i: (0, i, 0)),
            pl.BlockSpec((D, D), lambda i: (0, 0)),
            pl.BlockSpec((1, D), lambda i: (0, 0)),
        ],
        out_specs=pl.BlockSpec((_BLK, D), lambda i: (i, 0)),
        out_shape=jax.ShapeDtypeStruct((N_NODES, D), jnp.float32),
    )(parts, w, b2d)


@jax.jit
def kernel(edge_index, edge_weight, x, W1, b1, W2, b2):
    pad = ((0, 0), (0, EPAD - EPW))
    src = jnp.pad(edge_index[0].reshape(NW, EPW), pad)
    dst = jnp.pad(edge_index[1].reshape(NW, EPW), pad)
    wbits = lax.bitcast_convert_type(edge_weight, jnp.int32)
    wbits = jnp.pad(wbits.reshape(NW, EPW), pad)
    edata = jnp.stack([src.reshape(NW, NCHUNK, CH),
                       dst.reshape(NW, NCHUNK, CH),
                       wbits.reshape(NW, NCHUNK, CH)],
                      axis=2)
    zeros = jnp.zeros((N_NODES, D), jnp.float32)
    b1_2d = b1.reshape(1, D)
    b2_2d = b2.reshape(1, D)

    spmm = _make_spmm()
    m1 = spmm(x, edata, zeros)
    h = _mm_bias(m1, W1, b1_2d)
    m2 = spmm(h, edata, zeros)
    return _mm_bias(m2, W2, b2_2d)


# speedup vs baseline: 1.6745x; 1.6745x over previous
import functools

import jax
import jax.numpy as jnp
from jax import lax
from jax.experimental import pallas as pl
from jax.experimental.pallas import tpu as pltpu
from jax.experimental.pallas import tpu_sc as plsc

N_NODES = 10000
N_EDGES = 320000
D = 128

NC = 2
NS = 16
NW = NC * NS
EPW = N_EDGES // NW
CH = 125
NCHUNK = EPW // CH
RING = 3
NGRP = -(-NCHUNK // RING)
RPT = 624
RTAIL = N_NODES - NS * RPT


def _spmm_body(sup_hbm, edata_hbm, zeros_hbm, part_hbm,
               acc, ebuf, rows_v, sidx, esem, gsem, ssem):
    c = lax.axis_index("c")
    s = lax.axis_index("s")
    wid = s * NC + c
    my_edata = edata_hbm.at[wid]

    pltpu.sync_copy(zeros_hbm.at[pl.ds(s * RPT, RPT)],
                    acc.at[pl.ds(s * RPT, RPT)])

    @pl.when(s == NS - 1)
    def _zero_tail():
        tl = pl.ds(NS * RPT, RTAIL)
        pltpu.sync_copy(zeros_hbm.at[tl], acc.at[tl])

    plsc.subcore_barrier()

    def e_issue(cj, k):
        pltpu.async_copy(my_edata.at[cj], ebuf.at[pl.ds(3 * k, 3)],
                         esem.at[k])

    def e_wait(k):
        pltpu.make_async_copy(my_edata.at[0], ebuf.at[pl.ds(3 * k, 3)],
                              esem.at[k]).wait()

    def rows_slot(k):
        return rows_v.at[pl.ds(k * CH, CH)]

    def g_issue(k):
        pltpu.async_copy(sup_hbm.at[ebuf.at[3 * k]], rows_slot(k),
                         gsem.at[k])

    def g_wait(k):
        pltpu.make_async_copy(sup_hbm.at[ebuf.at[3 * k]], rows_slot(k),
                              gsem.at[k]).wait()

    _SIDX_OFFS = tuple(range(0, CH - 15, 16)) + (CH - 16,)

    def s_issue(k):
        for off in _SIDX_OFFS:
            sl = pl.ds(off, 16)
            sidx[k, sl] = ebuf[3 * k + 1, sl]
        pltpu.async_copy(rows_slot(k), acc.at[sidx.at[k]],
                         ssem.at[k], add=True)

    def s_wait(k):
        pltpu.make_async_copy(rows_slot(k), acc.at[sidx.at[k]],
                              ssem.at[k]).wait()

    def scale_slot(k):
        @plsc.parallel_loop(0, CH, unroll=5)
        def edge_body(e):
            wbits = plsc.load_gather(
                ebuf, [jnp.full((16,), 3 * k + 2, dtype=jnp.int32),
                       jnp.full((16,), e, dtype=jnp.int32)])
            wvec = plsc.bitcast(wbits, jnp.float32)
            row = k * CH + e
            for j in range(D // 16):
                sl = pl.ds(j * 16, 16)
                rows_v[row, sl] = rows_v[row, sl] * wvec

    e_issue(0, 0)
    e_issue(1, 1)
    e_wait(0)
    g_issue(0)

    def group_body(g, _):
        c0 = g * RING
        for k in range(RING):
            cj = c0 + k
            nb1 = (k + 1) % RING
            nb2 = (k + 2) % RING

            @pl.when(cj + 2 < NCHUNK)
            def _prefetch_edata():
                e_issue(cj + 2, nb2)

            @pl.when(cj + 1 < NCHUNK)
            def _prefetch_gather():
                @pl.when(cj + 1 >= RING)
                def _drain_scatter():
                    s_wait(nb1)
                e_wait(nb1)
                g_issue(nb1)

            @pl.when(cj < NCHUNK)
            def _process():
                g_wait(k)
                scale_slot(k)
                s_issue(k)
        return 0

    lax.fori_loop(0, NGRP, group_body, 0, unroll=False)
    for k in range(RING):
        s_wait(k)
    plsc.subcore_barrier()

    sl = pl.ds(s * RPT, RPT)
    pltpu.sync_copy(acc.at[sl], part_hbm.at[c].at[sl])

    @pl.when(s == NS - 1)
    def _write_tail():
        tl = pl.ds(NS * RPT, RTAIL)
        pltpu.sync_copy(acc.at[tl], part_hbm.at[c].at[tl])


@functools.lru_cache(maxsize=None)
def _make_spmm():
    return pl.kernel(
        _spmm_body,
        out_type=jax.ShapeDtypeStruct((NC, N_NODES, D), jnp.float32),
        mesh=plsc.VectorSubcoreMesh(core_axis_name="c", subcore_axis_name="s"),
        compiler_params=pltpu.CompilerParams(needs_layout_passes=False),
        scratch_types=[
            pltpu.VMEM_SHARED((N_NODES, D), jnp.float32),
            pltpu.VMEM((RING * 3, CH), jnp.int32),
            pltpu.VMEM((RING * CH, D), jnp.float32),
            pltpu.VMEM((RING, CH), jnp.int32),
            pltpu.SemaphoreType.DMA((RING,)),
            pltpu.SemaphoreType.DMA((RING,)),
            pltpu.SemaphoreType.DMA((RING,)),
        ],
    )


_BLK = 1000


def _mm_bias_kernel(p_ref, w_ref, b_ref, o_ref):
    o_ref[...] = jnp.dot(p_ref[0] + p_ref[1], w_ref[...],
                         preferred_element_type=jnp.float32,
                         precision=lax.Precision.HIGHEST) + b_ref[...]


def _mm_bias(parts, w, b2d):
    return pl.pallas_call(
        _mm_bias_kernel,
        grid=(N_NODES // _BLK,),
        in_specs=[
            pl.BlockSpec((NC, _BLK, D), lambda i: (0, i, 0)),
            pl.BlockSpec((D, D), lambda i: (0, 0)),
            pl.BlockSpec((1, D), lambda i: (0, 0)),
        ],
        out_specs=pl.BlockSpec((_BLK, D), lambda i: (i, 0)),
        out_shape=jax.ShapeDtypeStruct((N_NODES, D), jnp.float32),
    )(parts, w, b2d)


@jax.jit
def kernel(edge_index, edge_weight, x, W1, b1, W2, b2):
    src = edge_index[0].reshape(NW, NCHUNK, CH)
    dst = edge_index[1].reshape(NW, NCHUNK, CH)
    wbits = lax.bitcast_convert_type(edge_weight, jnp.int32)
    wbits = wbits.reshape(NW, NCHUNK, CH)
    edata = jnp.stack([src, dst, wbits], axis=2)
    zeros = jnp.zeros((N_NODES, D), jnp.float32)
    b1_2d = b1.reshape(1, D)
    b2_2d = b2.reshape(1, D)

    spmm = _make_spmm()
    m1 = spmm(x, edata, zeros)
    h = _mm_bias(m1, W1, b1_2d)
    m2 = spmm(h, edata, zeros)
    return _mm_bias(m2, W2, b2_2d)


# speedup vs baseline: 1.7037x; 1.0175x over previous
import functools

import jax
import jax.numpy as jnp
from jax import lax
from jax.experimental import pallas as pl
from jax.experimental.pallas import tpu as pltpu
from jax.experimental.pallas import tpu_sc as plsc

N_NODES = 10000
N_EDGES = 320000
D = 128

NC = 2
NS = 16
NW = NC * NS
EPW = N_EDGES // NW
CH = 125
NCHUNK = EPW // CH
RING = 3
NGRP = -(-NCHUNK // RING)
RPT = 624
RTAIL = N_NODES - NS * RPT


def _spmm_body(sup_hbm, edata_hbm, zeros_hbm, part_hbm,
               acc, ebuf, rows_v, sidx, esem, gsem, ssem, zsem):
    c = lax.axis_index("c")
    s = lax.axis_index("s")
    wid = s * NC + c
    my_edata = edata_hbm.at[wid]

    zsl = pl.ds(s * RPT, RPT)
    pltpu.async_copy(zeros_hbm.at[zsl], acc.at[zsl], zsem)

    @pl.when(s == NS - 1)
    def _zero_tail():
        tl = pl.ds(NS * RPT, RTAIL)
        pltpu.async_copy(zeros_hbm.at[tl], acc.at[tl], zsem)

    def e_issue(cj, k):
        pltpu.async_copy(my_edata.at[cj], ebuf.at[pl.ds(3 * k, 3)],
                         esem.at[k])

    def e_wait(k):
        pltpu.make_async_copy(my_edata.at[0], ebuf.at[pl.ds(3 * k, 3)],
                              esem.at[k]).wait()

    def rows_slot(k):
        return rows_v.at[pl.ds(k * CH, CH)]

    def g_issue(k):
        pltpu.async_copy(sup_hbm.at[ebuf.at[3 * k]], rows_slot(k),
                         gsem.at[k])

    def g_wait(k):
        pltpu.make_async_copy(sup_hbm.at[ebuf.at[3 * k]], rows_slot(k),
                              gsem.at[k]).wait()

    _SIDX_OFFS = tuple(range(0, CH - 15, 16)) + (CH - 16,)

    def s_issue(k):
        for off in _SIDX_OFFS:
            sl = pl.ds(off, 16)
            sidx[k, sl] = ebuf[3 * k + 1, sl]
        pltpu.async_copy(rows_slot(k), acc.at[sidx.at[k]],
                         ssem.at[k], add=True)

    def s_wait(k):
        pltpu.make_async_copy(rows_slot(k), acc.at[sidx.at[k]],
                              ssem.at[k]).wait()

    def scale_slot(k):
        @plsc.parallel_loop(0, CH, unroll=5)
        def edge_body(e):
            wbits = plsc.load_gather(
                ebuf, [jnp.full((16,), 3 * k + 2, dtype=jnp.int32),
                       jnp.full((16,), e, dtype=jnp.int32)])
            wvec = plsc.bitcast(wbits, jnp.float32)
            row = k * CH + e
            for j in range(D // 16):
                sl = pl.ds(j * 16, 16)
                rows_v[row, sl] = rows_v[row, sl] * wvec

    e_issue(0, 0)
    e_issue(1, 1)
    e_wait(0)
    g_issue(0)

    pltpu.make_async_copy(zeros_hbm.at[zsl], acc.at[zsl], zsem).wait()

    @pl.when(s == NS - 1)
    def _zero_tail_wait():
        tl = pl.ds(NS * RPT, RTAIL)
        pltpu.make_async_copy(zeros_hbm.at[tl], acc.at[tl], zsem).wait()

    plsc.subcore_barrier()

    def group_body(g, _):
        c0 = g * RING
        for k in range(RING):
            cj = c0 + k
            nb1 = (k + 1) % RING
            nb2 = (k + 2) % RING

            @pl.when(cj + 2 < NCHUNK)
            def _prefetch_edata():
                e_issue(cj + 2, nb2)

            @pl.when(cj + 1 < NCHUNK)
            def _prefetch_gather():
                @pl.when(cj + 1 >= RING)
                def _drain_scatter():
                    s_wait(nb1)
                e_wait(nb1)
                g_issue(nb1)

            @pl.when(cj < NCHUNK)
            def _process():
                g_wait(k)
                scale_slot(k)
                s_issue(k)
        return 0

    lax.fori_loop(0, NGRP, group_body, 0, unroll=False)
    for k in range(RING):
        s_wait(k)
    plsc.subcore_barrier()

    sl = pl.ds(s * RPT, RPT)
    pltpu.sync_copy(acc.at[sl], part_hbm.at[c].at[sl])

    @pl.when(s == NS - 1)
    def _write_tail():
        tl = pl.ds(NS * RPT, RTAIL)
        pltpu.sync_copy(acc.at[tl], part_hbm.at[c].at[tl])


@functools.lru_cache(maxsize=None)
def _make_spmm():
    return pl.kernel(
        _spmm_body,
        out_type=jax.ShapeDtypeStruct((NC, N_NODES, D), jnp.float32),
        mesh=plsc.VectorSubcoreMesh(core_axis_name="c", subcore_axis_name="s"),
        compiler_params=pltpu.CompilerParams(needs_layout_passes=False),
        scratch_types=[
            pltpu.VMEM_SHARED((N_NODES, D), jnp.float32),
            pltpu.VMEM((RING * 3, CH), jnp.int32),
            pltpu.VMEM((RING * CH, D), jnp.float32),
            pltpu.VMEM((RING, CH), jnp.int32),
            pltpu.SemaphoreType.DMA((RING,)),
            pltpu.SemaphoreType.DMA((RING,)),
            pltpu.SemaphoreType.DMA((RING,)),
            pltpu.SemaphoreType.DMA,
        ],
    )


_BLK = 1000


def _mm_bias_kernel(p_ref, w_ref, b_ref, o_ref):
    o_ref[...] = jnp.dot(p_ref[0] + p_ref[1], w_ref[...],
                         preferred_element_type=jnp.float32,
                         precision=lax.Precision.HIGHEST) + b_ref[...]


def _mm_bias(parts, w, b2d):
    return pl.pallas_call(
        _mm_bias_kernel,
        grid=(N_NODES // _BLK,),
        in_specs=[
            pl.BlockSpec((NC, _BLK, D), lambda i: (0, i, 0)),
            pl.BlockSpec((D, D), lambda i: (0, 0)),
            pl.BlockSpec((1, D), lambda i: (0, 0)),
        ],
        out_specs=pl.BlockSpec((_BLK, D), lambda i: (i, 0)),
        out_shape=jax.ShapeDtypeStruct((N_NODES, D), jnp.float32),
    )(parts, w, b2d)


@jax.jit
def kernel(edge_index, edge_weight, x, W1, b1, W2, b2):
    src = edge_index[0].reshape(NW, NCHUNK, CH)
    dst = edge_index[1].reshape(NW, NCHUNK, CH)
    wbits = lax.bitcast_convert_type(edge_weight, jnp.int32)
    wbits = wbits.reshape(NW, NCHUNK, CH)
    edata = jnp.stack([src, dst, wbits], axis=2)
    zeros = jnp.zeros((N_NODES, D), jnp.float32)
    b1_2d = b1.reshape(1, D)
    b2_2d = b2.reshape(1, D)

    spmm = _make_spmm()
    m1 = spmm(x, edata, zeros)
    h = _mm_bias(m1, W1, b1_2d)
    m2 = spmm(h, edata, zeros)
    return _mm_bias(m2, W2, b2_2d)
